# Initial kernel scaffold; baseline (speedup 1.0000x reference)
import jax
import jax.numpy as jnp
from jax.experimental import pallas as pl


def kernel(x, K, W, b1, b2):
    raise NotImplementedError("write your pallas kernel here")


# speedup vs baseline: 10.7861x; 10.7861x over previous
import jax
import jax.numpy as jnp
from jax import lax
from jax.experimental import pallas as pl
from jax.experimental.pallas import tpu as pltpu

_ROWS = 256


def _body(k_ref, x_ref, w_ref, b1_ref, b2_ref, enc_ref, dec_ref, nnz_ref,
          res_ref):
    enc = jnp.dot(x_ref[...], w_ref[...],
                  preferred_element_type=jnp.float32,
                  precision=lax.Precision.DEFAULT) + b1_ref[...]
    enc_ref[...] = enc
    a = jnp.abs(enc)
    u = lax.bitcast_convert_type(a, jnp.int32)
    k = k_ref[0]

    def bit_step(i, v):
        t = v | (jnp.int32(1) << (jnp.int32(30) - i))
        cnt = jnp.sum(jnp.where(u >= t, jnp.int32(1), jnp.int32(0)), axis=1,
                      keepdims=True)
        return jnp.where(cnt > k, t, v)

    v0 = jnp.zeros((u.shape[0], 1), jnp.int32)
    v = lax.fori_loop(0, 31, bit_step, v0)
    thr = lax.bitcast_convert_type(v, jnp.float32)

    keep = a > thr
    res = jnp.where(keep, enc, 0.0)
    res_ref[...] = res

    @pl.when(pl.program_id(0) == 0)
    def _init():
        nnz_ref[...] = jnp.zeros_like(nnz_ref)

    cnt2 = jnp.sum(jnp.where(keep, 1.0, 0.0), axis=1, keepdims=True)
    nnz_ref[...] += jnp.sum(cnt2, axis=0, keepdims=True)

    dec = lax.dot_general(res, w_ref[...], (((1,), (1,)), ((), ())),
                          preferred_element_type=jnp.float32,
                          precision=lax.Precision.DEFAULT) + b2_ref[...]
    dec_ref[...] = dec


def kernel(x, K, W, b1, b2):
    B, D = x.shape
    m = W.shape[1]
    rows = _ROWS if B % _ROWS == 0 else B
    grid = (B // rows,)
    k_arr = jnp.asarray(K, jnp.int32).reshape(1)
    enc, dec, nnz, res = pl.pallas_call(
        _body,
        grid=grid,
        in_specs=[
            pl.BlockSpec(memory_space=pltpu.SMEM),
            pl.BlockSpec((rows, D), lambda i: (i, 0)),
            pl.BlockSpec((D, m), lambda i: (0, 0)),
            pl.BlockSpec((1, m), lambda i: (0, 0)),
            pl.BlockSpec((1, D), lambda i: (0, 0)),
        ],
        out_specs=[
            pl.BlockSpec((rows, m), lambda i: (i, 0)),
            pl.BlockSpec((rows, D), lambda i: (i, 0)),
            pl.BlockSpec((1, 1), lambda i: (0, 0)),
            pl.BlockSpec((rows, m), lambda i: (i, 0)),
        ],
        out_shape=[
            jax.ShapeDtypeStruct((B, m), jnp.float32),
            jax.ShapeDtypeStruct((B, D), jnp.float32),
            jax.ShapeDtypeStruct((1, 1), jnp.float32),
            jax.ShapeDtypeStruct((B, m), jnp.float32),
        ],
    )(k_arr, x, W, b1, b2)
    return (enc, dec, (nnz[0, 0] / B).astype(jnp.float32), res)
